# baseline (device time: 17329 ns/iter reference)
import jax
import jax.numpy as jnp
from jax import lax
from jax.experimental import pallas as pl
from jax.experimental.pallas import tpu as pltpu

X_DEV = 2


def kernel(x):
    m_per, n = x.shape

    def body(x_ref, out_ref, local_sem, send_sem, recv_sem, send_sem2, recv_sem2):
        my_x = lax.axis_index("x")
        my_y = lax.axis_index("y")
        my_z = lax.axis_index("z")
        peer = (1 - my_x, my_y, my_z)

        barrier_sem = pltpu.get_barrier_semaphore()
        pl.semaphore_signal(
            barrier_sem, inc=1, device_id=peer,
            device_id_type=pl.DeviceIdType.MESH,
        )
        pl.semaphore_wait(barrier_sem, 1)

        half = m_per // 2
        rdma0 = pltpu.make_async_remote_copy(
            src_ref=x_ref.at[pl.ds(0, half)],
            dst_ref=out_ref.at[pl.ds(my_x * m_per, half)],
            send_sem=send_sem,
            recv_sem=recv_sem,
            device_id=peer,
            device_id_type=pl.DeviceIdType.MESH,
        )
        rdma1 = pltpu.make_async_remote_copy(
            src_ref=x_ref.at[pl.ds(half, half)],
            dst_ref=out_ref.at[pl.ds(my_x * m_per + half, half)],
            send_sem=send_sem2,
            recv_sem=recv_sem2,
            device_id=peer,
            device_id_type=pl.DeviceIdType.MESH,
        )
        rdma0.start()
        rdma1.start()

        local = pltpu.make_async_copy(
            x_ref,
            out_ref.at[pl.ds(my_x * m_per, m_per)],
            local_sem,
        )
        local.start()
        local.wait()
        rdma0.wait()
        rdma1.wait()

    return pl.pallas_call(
        body,
        out_shape=jax.ShapeDtypeStruct((X_DEV * m_per, n), x.dtype),
        in_specs=[pl.BlockSpec(memory_space=pltpu.VMEM)],
        out_specs=pl.BlockSpec(memory_space=pltpu.MemorySpace.HBM),
        scratch_shapes=[
            pltpu.SemaphoreType.DMA,
            pltpu.SemaphoreType.DMA,
            pltpu.SemaphoreType.DMA,
            pltpu.SemaphoreType.DMA,
            pltpu.SemaphoreType.DMA,
        ],
        compiler_params=pltpu.CompilerParams(collective_id=0),
    )(x)
